# Initial kernel scaffold; baseline (speedup 1.0000x reference)
import jax
import jax.numpy as jnp
from jax.experimental import pallas as pl


def kernel(x, router_w, W1, b1, W2, b2, W3, b3, W4, b4):
    raise NotImplementedError("write your pallas kernel here")


# speedup vs baseline: 2.4629x; 2.4629x over previous
import functools

import jax
import jax.numpy as jnp
from jax.experimental import pallas as pl
from jax.experimental.pallas import tpu as pltpu

S = 1024
D = 1024
E = 8
K = 2
C = 64
NTOT = 2 * S
BS = 920
N_REAL = 2 * BS
TI = 256
GSTEPS = NTOT // TI
HP = jax.lax.Precision.HIGHEST


def _route_body(x_ref, rwt_ref, noise_ref, c0_ref, c1_ref, d0_ref, d1_ref,
                bal_ref, ei0_ref, ei1_ref):
    x = x_ref[...]
    select = jax.lax.dot_general(x, rwt_ref[...], (((1,), (0,)), ((), ())),
                                 precision=HP) + noise_ref[...]
    lane = jax.lax.broadcasted_iota(jnp.int32, (S, E), 1)
    g0 = jnp.max(select, axis=1, keepdims=True)
    i0 = jnp.min(jnp.where(select == g0, lane, E), axis=1, keepdims=True)
    masked = jnp.where(lane == i0, -jnp.inf, select)
    g1 = jnp.max(masked, axis=1, keepdims=True)
    i1 = jnp.min(jnp.where(masked == g1, lane, E), axis=1, keepdims=True)
    m0 = (lane == i0).astype(jnp.float32)
    m1 = (lane == i1).astype(jnp.float32)
    c0 = g0 * m0
    c1 = g1 * m1
    d0 = (c0 != 0.0).astype(jnp.float32)
    d1 = (c1 != 0.0).astype(jnp.float32)
    c0_ref[...] = c0
    c1_ref[...] = c1
    d0_ref[...] = d0
    d1_ref[...] = d1
    density_colsum = jnp.sum((m0 + m1) * 0.5, axis=0, keepdims=True)
    proxy = jnp.sum(select, axis=0, keepdims=True) * (1.0 / S)
    bal = jnp.sum(proxy * density_colsum) * (float(E * E) / (S * E))
    bal_ref[...] = jnp.broadcast_to(bal, (1, 1))
    ei0_ref[...] = jax.lax.dot_general(d0, x, (((0,), (0,)), ((), ())),
                                       precision=HP)
    ei1_ref[...] = jax.lax.dot_general(d1, x, (((0,), (0,)), ((), ())),
                                       precision=HP)


def _experts_body(ei_ref, w1_ref, b1_ref, w2_ref, b2_ref, w3_ref, b3_ref,
                  w4_ref, b4_ref, o_ref):
    inp = ei_ref[0]
    h = jax.lax.dot_general(inp, w1_ref[0], (((1,), (1,)), ((), ())),
                            precision=HP) + b1_ref[0]
    h = jnp.maximum(h, 0.0)
    h = jax.lax.dot_general(h, w2_ref[0], (((1,), (1,)), ((), ())),
                            precision=HP) + b2_ref[0]
    h = jnp.maximum(h, 0.0)
    h = jax.lax.dot_general(h, w3_ref[0], (((1,), (1,)), ((), ())),
                            precision=HP) + b3_ref[0]
    h = jnp.maximum(h, 0.0)
    o_ref[0] = jax.lax.dot_general(h, w4_ref[0], (((1,), (1,)), ((), ())),
                                   precision=HP) + b4_ref[0]


def _mmd_body(c0_ref, c1_ref, o0_ref, o1_ref, scol_ref, mcol_ref,
              out_ref, dist_ref, a_scr, bt_scr, coef_scr, acc_scr):
    g = pl.program_id(0)

    @pl.when(g == 0)
    def _prep():
        mid0 = jax.lax.dot_general(c0_ref[...], o0_ref[...],
                                   (((1,), (0,)), ((), ())), precision=HP)
        mid1 = jax.lax.dot_general(c1_ref[...], o1_ref[...],
                                   (((1,), (0,)), ((), ())), precision=HP)
        out_ref[...] = mid0 + mid1
        n0 = jnp.sum(mid0 * mid0, axis=1, keepdims=True)
        n1 = jnp.sum(mid1 * mid1, axis=1, keepdims=True)
        ones = jnp.ones((S, 1), jnp.float32)
        zpad = jnp.zeros((S, 128 - C - 2), jnp.float32)
        a_scr[0:S, :] = jnp.concatenate([-2.0 * mid0, n0, ones, zpad], axis=1)
        a_scr[S:NTOT, :] = jnp.concatenate([-2.0 * mid1, n1, ones, zpad],
                                           axis=1)
        bt_scr[0:S, :] = jnp.concatenate([mid0, ones, n0, zpad], axis=1)
        bt_scr[S:NTOT, :] = jnp.concatenate([mid1, ones, n1, zpad], axis=1)
        m0c = mcol_ref[0:S, :]
        m1c = mcol_ref[S:NTOT, :]
        v = (jax.lax.dot_general(mid0, m0c, (((0,), (0,)), ((), ())),
                                 precision=HP)
             + jax.lax.dot_general(mid1, m1c, (((0,), (0,)), ((), ())),
                                   precision=HP))
        ssq = jnp.sum(v * v)
        sum_mn = jnp.sum(m0c * n0) + jnp.sum(m1c * n1)
        sum_l2 = 2.0 * N_REAL * sum_mn - 2.0 * ssq
        bw = sum_l2 / float(N_REAL * N_REAL - N_REAL)
        b0 = bw * 0.25
        coef_scr[0, 0] = 1.0 / (16.0 * b0)
        acc_scr[0, 0] = 0.0

    ablk = a_scr[pl.ds(g * TI, TI), :]
    l2 = jax.lax.dot_general(ablk, bt_scr[...], (((1,), (1,)), ((), ())),
                             precision=HP)
    u = jnp.exp(-l2 * coef_scr[0, 0])
    u2 = u * u
    u4 = u2 * u2
    u8 = u4 * u4
    p = u + u2 + u4 + u8 + u8 * u8
    r = jax.lax.dot_general(p, scol_ref[...], (((1,), (0,)), ((), ())),
                            precision=HP)
    acc_scr[0, 0] += jnp.sum(r * scol_ref[pl.ds(g * TI, TI), :])

    @pl.when(g == GSTEPS - 1)
    def _fin():
        dist_ref[...] = jnp.broadcast_to(-acc_scr[0, 0] / float(BS * BS),
                                         (1, 1))


@functools.partial(jax.jit, static_argnums=())
def kernel(x, router_w, W1, b1, W2, b2, W3, b3, W4, b4):
    f32 = jnp.float32
    noise = jax.random.uniform(jax.random.key(1), (S, E), dtype=f32)
    s1 = jax.random.permutation(jax.random.fold_in(jax.random.key(2), 0), S)[:BS]
    s2 = jax.random.permutation(jax.random.fold_in(jax.random.key(2), 1), S)[:BS]
    w_src = jnp.zeros((S,), f32).at[s1].set(1.0)
    w_tgt = jnp.zeros((S,), f32).at[s2].set(1.0)
    m_col = jnp.concatenate([w_src, w_tgt]).reshape(NTOT, 1)
    s_col = jnp.concatenate([w_src, -w_tgt]).reshape(NTOT, 1)

    c0, c1, d0, d1, bal, ei0, ei1 = pl.pallas_call(
        _route_body,
        out_shape=[
            jax.ShapeDtypeStruct((S, E), f32),
            jax.ShapeDtypeStruct((S, E), f32),
            jax.ShapeDtypeStruct((S, E), f32),
            jax.ShapeDtypeStruct((S, E), f32),
            jax.ShapeDtypeStruct((1, 1), f32),
            jax.ShapeDtypeStruct((E, D), f32),
            jax.ShapeDtypeStruct((E, D), f32),
        ],
    )(x, router_w.T, noise)

    ei = jnp.stack([ei0, ei1], axis=1)
    outs = pl.pallas_call(
        _experts_body,
        grid=(E,),
        in_specs=[
            pl.BlockSpec((1, K, D), lambda e: (e, 0, 0)),
            pl.BlockSpec((1, 500, D), lambda e: (e, 0, 0)),
            pl.BlockSpec((1, 1, 500), lambda e: (e, 0, 0)),
            pl.BlockSpec((1, 500, 500), lambda e: (e, 0, 0)),
            pl.BlockSpec((1, 1, 500), lambda e: (e, 0, 0)),
            pl.BlockSpec((1, 2000, 500), lambda e: (e, 0, 0)),
            pl.BlockSpec((1, 1, 2000), lambda e: (e, 0, 0)),
            pl.BlockSpec((1, C, 2000), lambda e: (e, 0, 0)),
            pl.BlockSpec((1, 1, C), lambda e: (e, 0, 0)),
        ],
        out_specs=pl.BlockSpec((1, K, C), lambda e: (e, 0, 0)),
        out_shape=jax.ShapeDtypeStruct((E, K, C), f32),
    )(ei, W1, b1.reshape(E, 1, 500), W2, b2.reshape(E, 1, 500),
      W3, b3.reshape(E, 1, 2000), W4, b4.reshape(E, 1, C))

    o0 = outs[:, 0, :]
    o1 = outs[:, 1, :]
    out, dist = pl.pallas_call(
        _mmd_body,
        grid=(GSTEPS,),
        in_specs=[
            pl.BlockSpec((S, E), lambda g: (0, 0)),
            pl.BlockSpec((S, E), lambda g: (0, 0)),
            pl.BlockSpec((E, C), lambda g: (0, 0)),
            pl.BlockSpec((E, C), lambda g: (0, 0)),
            pl.BlockSpec((NTOT, 1), lambda g: (0, 0)),
            pl.BlockSpec((NTOT, 1), lambda g: (0, 0)),
        ],
        out_specs=[
            pl.BlockSpec((S, C), lambda g: (0, 0)),
            pl.BlockSpec((1, 1), lambda g: (0, 0)),
        ],
        out_shape=[
            jax.ShapeDtypeStruct((S, C), f32),
            jax.ShapeDtypeStruct((1, 1), f32),
        ],
        scratch_shapes=[
            pltpu.VMEM((NTOT, 128), f32),
            pltpu.VMEM((NTOT, 128), f32),
            pltpu.SMEM((1, 1), f32),
            pltpu.SMEM((1, 1), f32),
        ],
        compiler_params=pltpu.CompilerParams(
            dimension_semantics=("arbitrary",)),
    )(c0, c1, o0, o1, s_col, m_col)

    select0 = jnp.stack([d0, d1], axis=-1)
    return (out, select0, bal.reshape(()), dist.reshape(()))


# speedup vs baseline: 3.3907x; 1.3767x over previous
import functools

import jax
import jax.numpy as jnp
from jax.experimental import pallas as pl
from jax.experimental.pallas import tpu as pltpu

S = 1024
D = 1024
E = 8
K = 2
C = 64
NTOT = 2 * S
BS = 920
N_REAL = 2 * BS
TI = 256
GSTEPS = NTOT // TI
H3 = 1000
HP = jax.lax.Precision.HIGHEST


def _route_body(x_ref, rwt_ref, noise_ref, c0_ref, c1_ref, d0_ref, d1_ref,
                bal_ref, ei_ref):
    x = x_ref[...]
    select = jax.lax.dot_general(x, rwt_ref[...], (((1,), (0,)), ((), ()))
                                 ) + noise_ref[...]
    lane = jax.lax.broadcasted_iota(jnp.int32, (S, E), 1)
    g0 = jnp.max(select, axis=1, keepdims=True)
    i0 = jnp.min(jnp.where(select == g0, lane, E), axis=1, keepdims=True)
    masked = jnp.where(lane == i0, -jnp.inf, select)
    g1 = jnp.max(masked, axis=1, keepdims=True)
    i1 = jnp.min(jnp.where(masked == g1, lane, E), axis=1, keepdims=True)
    m0 = (lane == i0).astype(jnp.float32)
    m1 = (lane == i1).astype(jnp.float32)
    c0 = g0 * m0
    c1 = g1 * m1
    d0 = (c0 != 0.0).astype(jnp.float32)
    d1 = (c1 != 0.0).astype(jnp.float32)
    c0_ref[...] = c0
    c1_ref[...] = c1
    d0_ref[...] = d0
    d1_ref[...] = d1
    density_colsum = jnp.sum((m0 + m1) * 0.5, axis=0, keepdims=True)
    proxy = jnp.sum(select, axis=0, keepdims=True) * (1.0 / S)
    bal = jnp.sum(proxy * density_colsum) * (float(E * E) / (S * E))
    bal_ref[...] = jnp.broadcast_to(bal, (1, 1))
    d01 = jnp.concatenate([d0, d1], axis=1)
    ei_ref[...] = jax.lax.dot_general(d01, x, (((0,), (0,)), ((), ())))


def _experts_body(ei_ref, w1_ref, b1_ref, w2_ref, b2_ref, w3_ref, b3_ref,
                  w4t_ref, b4_ref, o_ref, h2_scr):
    j = pl.program_id(1)

    @pl.when(j == 0)
    def _front():
        inp = ei_ref[0]
        h = jax.lax.dot_general(inp, w1_ref[0], (((1,), (1,)), ((), ()))
                                ) + b1_ref[0]
        h = jnp.maximum(h, 0.0)
        h = jax.lax.dot_general(h, w2_ref[0], (((1,), (1,)), ((), ()))
                                ) + b2_ref[0]
        h2_scr[...] = jnp.maximum(h, 0.0)

    h3 = jax.lax.dot_general(h2_scr[...], w3_ref[0, 0],
                             (((1,), (1,)), ((), ()))) + b3_ref[0, 0]
    h3 = jnp.maximum(h3, 0.0)
    part = jax.lax.dot_general(h3, w4t_ref[0, 0], (((1,), (0,)), ((), ())))

    @pl.when(j == 0)
    def _init():
        o_ref[0] = part + b4_ref[0]

    @pl.when(j == 1)
    def _acc():
        o_ref[0] = o_ref[0] + part


def _mmd_body(c0_ref, c1_ref, o0_ref, o1_ref, scol_ref, srow_ref, mcol_ref,
              out_ref, dist_ref, a_scr, bt_scr, coef_scr, acc_scr):
    g = pl.program_id(0)

    @pl.when(g == 0)
    def _prep():
        mid0 = jax.lax.dot_general(c0_ref[...], o0_ref[...],
                                   (((1,), (0,)), ((), ())))
        mid1 = jax.lax.dot_general(c1_ref[...], o1_ref[...],
                                   (((1,), (0,)), ((), ())))
        out_ref[...] = mid0 + mid1
        n0 = jnp.sum(mid0 * mid0, axis=1, keepdims=True)
        n1 = jnp.sum(mid1 * mid1, axis=1, keepdims=True)
        ones = jnp.ones((S, 1), jnp.float32)
        zpad = jnp.zeros((S, 128 - C - 2), jnp.float32)
        a_scr[0:S, :] = jnp.concatenate([-2.0 * mid0, n0, ones, zpad], axis=1)
        a_scr[S:NTOT, :] = jnp.concatenate([-2.0 * mid1, n1, ones, zpad],
                                           axis=1)
        bt_scr[0:S, :] = jnp.concatenate([mid0, ones, n0, zpad], axis=1)
        bt_scr[S:NTOT, :] = jnp.concatenate([mid1, ones, n1, zpad], axis=1)
        m0c = mcol_ref[0:S, :]
        m1c = mcol_ref[S:NTOT, :]
        v = (jnp.sum(mid0 * m0c, axis=0, keepdims=True)
             + jnp.sum(mid1 * m1c, axis=0, keepdims=True))
        ssq = jnp.sum(v * v)
        sum_mn = jnp.sum(m0c * n0) + jnp.sum(m1c * n1)
        sum_l2 = 2.0 * N_REAL * sum_mn - 2.0 * ssq
        bw = sum_l2 / float(N_REAL * N_REAL - N_REAL)
        b0 = bw * 0.25
        coef_scr[0, 0] = 1.0 / (16.0 * b0)
        acc_scr[0, 0] = 0.0

    ablk = a_scr[pl.ds(g * TI, TI), :]
    l2 = jax.lax.dot_general(ablk, bt_scr[...], (((1,), (1,)), ((), ())),
                             precision=HP)
    u = jnp.exp(-l2 * coef_scr[0, 0])
    u2 = u * u
    u4 = u2 * u2
    u8 = u4 * u4
    p = u + u2 + u4 + u8 + u8 * u8
    rs = jnp.sum(p * srow_ref[...], axis=1, keepdims=True)
    acc_scr[0, 0] += jnp.sum(rs * scol_ref[pl.ds(g * TI, TI), :])

    @pl.when(g == GSTEPS - 1)
    def _fin():
        dist_ref[...] = jnp.broadcast_to(-acc_scr[0, 0] / float(BS * BS),
                                         (1, 1))


@functools.partial(jax.jit, static_argnums=())
def kernel(x, router_w, W1, b1, W2, b2, W3, b3, W4, b4):
    f32 = jnp.float32
    noise = jax.random.uniform(jax.random.key(1), (S, E), dtype=f32)
    s1 = jax.random.permutation(jax.random.fold_in(jax.random.key(2), 0), S)[:BS]
    s2 = jax.random.permutation(jax.random.fold_in(jax.random.key(2), 1), S)[:BS]
    w_src = jnp.zeros((S,), f32).at[s1].set(1.0)
    w_tgt = jnp.zeros((S,), f32).at[s2].set(1.0)
    m_col = jnp.concatenate([w_src, w_tgt]).reshape(NTOT, 1)
    s_flat = jnp.concatenate([w_src, -w_tgt])
    s_col = s_flat.reshape(NTOT, 1)
    s_row = s_flat.reshape(1, NTOT)

    c0, c1, d0, d1, bal, ei = pl.pallas_call(
        _route_body,
        out_shape=[
            jax.ShapeDtypeStruct((S, E), f32),
            jax.ShapeDtypeStruct((S, E), f32),
            jax.ShapeDtypeStruct((S, E), f32),
            jax.ShapeDtypeStruct((S, E), f32),
            jax.ShapeDtypeStruct((1, 1), f32),
            jax.ShapeDtypeStruct((2 * E, D), f32),
        ],
    )(x, router_w.T, noise)

    eik = jnp.stack([ei[:E], ei[E:]], axis=1)
    outs = pl.pallas_call(
        _experts_body,
        grid=(E, 2),
        in_specs=[
            pl.BlockSpec((1, K, D), lambda e, j: (e, 0, 0)),
            pl.BlockSpec((1, 500, D), lambda e, j: (e, 0, 0)),
            pl.BlockSpec((1, 1, 500), lambda e, j: (e, 0, 0)),
            pl.BlockSpec((1, 500, 500), lambda e, j: (e, 0, 0)),
            pl.BlockSpec((1, 1, 500), lambda e, j: (e, 0, 0)),
            pl.BlockSpec((1, 1, H3, 500), lambda e, j: (e, j, 0, 0)),
            pl.BlockSpec((1, 1, 1, H3), lambda e, j: (e, j, 0, 0)),
            pl.BlockSpec((1, 1, H3, C), lambda e, j: (e, j, 0, 0)),
            pl.BlockSpec((1, 1, C), lambda e, j: (e, 0, 0)),
        ],
        out_specs=pl.BlockSpec((1, K, C), lambda e, j: (e, 0, 0)),
        out_shape=jax.ShapeDtypeStruct((E, K, C), f32),
        scratch_shapes=[pltpu.VMEM((K, 500), f32)],
        compiler_params=pltpu.CompilerParams(
            dimension_semantics=("arbitrary", "arbitrary")),
    )(eik, W1, b1.reshape(E, 1, 500), W2, b2.reshape(E, 1, 500),
      W3.reshape(E, 2, H3, 500), b3.reshape(E, 2, 1, H3),
      jnp.swapaxes(W4, 1, 2).reshape(E, 2, H3, C), b4.reshape(E, 1, C))

    o0 = outs[:, 0, :]
    o1 = outs[:, 1, :]
    out, dist = pl.pallas_call(
        _mmd_body,
        grid=(GSTEPS,),
        in_specs=[
            pl.BlockSpec((S, E), lambda g: (0, 0)),
            pl.BlockSpec((S, E), lambda g: (0, 0)),
            pl.BlockSpec((E, C), lambda g: (0, 0)),
            pl.BlockSpec((E, C), lambda g: (0, 0)),
            pl.BlockSpec((NTOT, 1), lambda g: (0, 0)),
            pl.BlockSpec((1, NTOT), lambda g: (0, 0)),
            pl.BlockSpec((NTOT, 1), lambda g: (0, 0)),
        ],
        out_specs=[
            pl.BlockSpec((S, C), lambda g: (0, 0)),
            pl.BlockSpec((1, 1), lambda g: (0, 0)),
        ],
        out_shape=[
            jax.ShapeDtypeStruct((S, C), f32),
            jax.ShapeDtypeStruct((1, 1), f32),
        ],
        scratch_shapes=[
            pltpu.VMEM((NTOT, 128), f32),
            pltpu.VMEM((NTOT, 128), f32),
            pltpu.SMEM((1, 1), f32),
            pltpu.SMEM((1, 1), f32),
        ],
        compiler_params=pltpu.CompilerParams(
            dimension_semantics=("arbitrary",)),
    )(c0, c1, o0, o1, s_col, s_row, m_col)

    select0 = jnp.stack([d0, d1], axis=-1)
    return (out, select0, bal.reshape(()), dist.reshape(()))


# speedup vs baseline: 4.2264x; 1.2465x over previous
import functools

import jax
import jax.numpy as jnp
import numpy as np
from jax.experimental import pallas as pl
from jax.experimental.pallas import tpu as pltpu

S = 1024
D = 1024
E = 8
K = 2
C = 64
NTOT = 2 * S
BS = 920
N_REAL = 2 * BS
TI = 256
GSTEPS = NTOT // TI
H3 = 1000
HP = jax.lax.Precision.HIGHEST


def _const_parts(use_numpy):
    noise = jax.random.uniform(jax.random.key(1), (S, E), dtype=jnp.float32)
    k1 = jax.random.fold_in(jax.random.key(2), 0)
    k2 = jax.random.fold_in(jax.random.key(2), 1)
    s1 = jax.random.permutation(k1, S)[:BS]
    s2 = jax.random.permutation(k2, S)[:BS]
    if use_numpy:
        s1, s2 = np.asarray(s1), np.asarray(s2)
        w_src = np.zeros((S,), np.float32)
        w_src[s1] = 1.0
        w_tgt = np.zeros((S,), np.float32)
        w_tgt[s2] = 1.0
        m_flat = np.concatenate([w_src, w_tgt])
        s_flat = np.concatenate([w_src, -w_tgt])
        return np.asarray(noise), m_flat, s_flat
    w_src = jnp.zeros((S,), jnp.float32).at[s1].set(1.0)
    w_tgt = jnp.zeros((S,), jnp.float32).at[s2].set(1.0)
    m_flat = jnp.concatenate([w_src, w_tgt])
    s_flat = jnp.concatenate([w_src, -w_tgt])
    return noise, m_flat, s_flat


def _host_constants():
    try:
        cpu = jax.devices("cpu")[0]
        with jax.default_device(cpu):
            return _const_parts(use_numpy=True)
    except Exception:
        return None


_CONSTS = _host_constants()


def _route_body(x_ref, rwt_ref, noise_ref, c0_ref, c1_ref, d0_ref, d1_ref,
                bal_ref, ei_ref):
    x = x_ref[...]
    select = jax.lax.dot_general(x, rwt_ref[...], (((1,), (0,)), ((), ()))
                                 ) + noise_ref[...]
    lane = jax.lax.broadcasted_iota(jnp.int32, (S, E), 1)
    g0 = jnp.max(select, axis=1, keepdims=True)
    i0 = jnp.min(jnp.where(select == g0, lane, E), axis=1, keepdims=True)
    masked = jnp.where(lane == i0, -jnp.inf, select)
    g1 = jnp.max(masked, axis=1, keepdims=True)
    i1 = jnp.min(jnp.where(masked == g1, lane, E), axis=1, keepdims=True)
    m0 = (lane == i0).astype(jnp.float32)
    m1 = (lane == i1).astype(jnp.float32)
    c0 = g0 * m0
    c1 = g1 * m1
    d0 = (c0 != 0.0).astype(jnp.float32)
    d1 = (c1 != 0.0).astype(jnp.float32)
    c0_ref[...] = c0
    c1_ref[...] = c1
    d0_ref[...] = d0
    d1_ref[...] = d1
    density_colsum = jnp.sum((m0 + m1) * 0.5, axis=0, keepdims=True)
    proxy = jnp.sum(select, axis=0, keepdims=True) * (1.0 / S)
    bal = jnp.sum(proxy * density_colsum) * (float(E * E) / (S * E))
    bal_ref[...] = jnp.broadcast_to(bal, (1, 1))
    d01 = jnp.concatenate([d0, d1], axis=1)
    ei_ref[...] = jax.lax.dot_general(d01, x, (((0,), (0,)), ((), ())))


def _experts_body(ei_ref, w1_ref, b1_ref, w2_ref, b2_ref, w3_ref, b3_ref,
                  w4_ref, b4_ref, o_ref, h2_scr, h3_scr):
    j = pl.program_id(1)

    @pl.when(j == 0)
    def _front():
        inp = ei_ref[0]
        h = jax.lax.dot_general(inp, w1_ref[0], (((1,), (1,)), ((), ()))
                                ) + b1_ref[0]
        h = jnp.maximum(h, 0.0)
        h = jax.lax.dot_general(h, w2_ref[0], (((1,), (1,)), ((), ()))
                                ) + b2_ref[0]
        h2_scr[...] = jnp.maximum(h, 0.0)

    h3 = jax.lax.dot_general(h2_scr[...], w3_ref[0, 0],
                             (((1,), (1,)), ((), ()))) + b3_ref[0, 0]
    h3 = jnp.maximum(h3, 0.0)

    @pl.when(j == 0)
    def _store0():
        h3_scr[:, 0:H3] = h3

    @pl.when(j == 1)
    def _store1():
        h3_scr[:, H3:2 * H3] = h3
        o_ref[0] = jax.lax.dot_general(h3_scr[...], w4_ref[0],
                                       (((1,), (1,)), ((), ()))) + b4_ref[0]


def _mmd_body(c0_ref, c1_ref, o0_ref, o1_ref, scol_ref, srow_ref, mcol_ref,
              out_ref, dist_ref, a_scr, bt_scr, coef_scr, acc_scr):
    g = pl.program_id(0)

    @pl.when(g == 0)
    def _prep():
        mid0 = jax.lax.dot_general(c0_ref[...], o0_ref[...],
                                   (((1,), (0,)), ((), ())))
        mid1 = jax.lax.dot_general(c1_ref[...], o1_ref[...],
                                   (((1,), (0,)), ((), ())))
        out_ref[...] = mid0 + mid1
        n0 = jnp.sum(mid0 * mid0, axis=1, keepdims=True)
        n1 = jnp.sum(mid1 * mid1, axis=1, keepdims=True)
        ones = jnp.ones((S, 1), jnp.float32)
        zpad = jnp.zeros((S, 128 - C - 2), jnp.float32)
        a_scr[0:S, :] = jnp.concatenate([-2.0 * mid0, n0, ones, zpad], axis=1)
        a_scr[S:NTOT, :] = jnp.concatenate([-2.0 * mid1, n1, ones, zpad],
                                           axis=1)
        bt_scr[0:S, :] = jnp.concatenate([mid0, ones, n0, zpad], axis=1)
        bt_scr[S:NTOT, :] = jnp.concatenate([mid1, ones, n1, zpad], axis=1)
        m0c = mcol_ref[0:S, :]
        m1c = mcol_ref[S:NTOT, :]
        v = (jnp.sum(mid0 * m0c, axis=0, keepdims=True)
             + jnp.sum(mid1 * m1c, axis=0, keepdims=True))
        ssq = jnp.sum(v * v)
        sum_mn = jnp.sum(m0c * n0) + jnp.sum(m1c * n1)
        sum_l2 = 2.0 * N_REAL * sum_mn - 2.0 * ssq
        bw = sum_l2 / float(N_REAL * N_REAL - N_REAL)
        b0 = bw * 0.25
        coef_scr[0, 0] = 1.0 / (16.0 * b0)
        acc_scr[0, 0] = 0.0

    ablk = a_scr[pl.ds(g * TI, TI), :]
    l2 = jax.lax.dot_general(ablk, bt_scr[...], (((1,), (1,)), ((), ())),
                             precision=HP)
    u = jnp.exp(-l2 * coef_scr[0, 0])
    u2 = u * u
    u4 = u2 * u2
    u8 = u4 * u4
    p = u + u2 + u4 + u8 + u8 * u8
    rs = jnp.sum(p * srow_ref[...], axis=1, keepdims=True)
    acc_scr[0, 0] += jnp.sum(rs * scol_ref[pl.ds(g * TI, TI), :])

    @pl.when(g == GSTEPS - 1)
    def _fin():
        dist_ref[...] = jnp.broadcast_to(-acc_scr[0, 0] / float(BS * BS),
                                         (1, 1))


@functools.partial(jax.jit, static_argnums=())
def kernel(x, router_w, W1, b1, W2, b2, W3, b3, W4, b4):
    f32 = jnp.float32
    noise, m_flat, s_flat = (_CONSTS if _CONSTS is not None
                             else _const_parts(use_numpy=False))
    noise = jnp.asarray(noise)
    m_col = jnp.asarray(m_flat).reshape(NTOT, 1)
    s_col = jnp.asarray(s_flat).reshape(NTOT, 1)
    s_row = jnp.asarray(s_flat).reshape(1, NTOT)

    c0, c1, d0, d1, bal, ei = pl.pallas_call(
        _route_body,
        out_shape=[
            jax.ShapeDtypeStruct((S, E), f32),
            jax.ShapeDtypeStruct((S, E), f32),
            jax.ShapeDtypeStruct((S, E), f32),
            jax.ShapeDtypeStruct((S, E), f32),
            jax.ShapeDtypeStruct((1, 1), f32),
            jax.ShapeDtypeStruct((2 * E, D), f32),
        ],
    )(x, router_w.T, noise)

    eik = jnp.stack([ei[:E], ei[E:]], axis=1)
    outs = pl.pallas_call(
        _experts_body,
        grid=(E, 2),
        in_specs=[
            pl.BlockSpec((1, K, D), lambda e, j: (e, 0, 0)),
            pl.BlockSpec((1, 500, D), lambda e, j: (e, 0, 0)),
            pl.BlockSpec((1, 1, 500), lambda e, j: (e, 0, 0)),
            pl.BlockSpec((1, 500, 500), lambda e, j: (e, 0, 0)),
            pl.BlockSpec((1, 1, 500), lambda e, j: (e, 0, 0)),
            pl.BlockSpec((1, 1, H3, 500), lambda e, j: (e, j, 0, 0)),
            pl.BlockSpec((1, 1, 1, H3), lambda e, j: (e, j, 0, 0)),
            pl.BlockSpec((1, C, 2 * H3), lambda e, j: (e, 0, 0)),
            pl.BlockSpec((1, 1, C), lambda e, j: (e, 0, 0)),
        ],
        out_specs=pl.BlockSpec((1, K, C), lambda e, j: (e, 0, 0)),
        out_shape=jax.ShapeDtypeStruct((E, K, C), f32),
        scratch_shapes=[pltpu.VMEM((K, 500), f32),
                        pltpu.VMEM((K, 2 * H3), f32)],
        compiler_params=pltpu.CompilerParams(
            dimension_semantics=("arbitrary", "arbitrary")),
    )(eik, W1, b1.reshape(E, 1, 500), W2, b2.reshape(E, 1, 500),
      W3.reshape(E, 2, H3, 500), b3.reshape(E, 2, 1, H3),
      W4, b4.reshape(E, 1, C))

    o0 = outs[:, 0, :]
    o1 = outs[:, 1, :]
    out, dist = pl.pallas_call(
        _mmd_body,
        grid=(GSTEPS,),
        in_specs=[
            pl.BlockSpec((S, E), lambda g: (0, 0)),
            pl.BlockSpec((S, E), lambda g: (0, 0)),
            pl.BlockSpec((E, C), lambda g: (0, 0)),
            pl.BlockSpec((E, C), lambda g: (0, 0)),
            pl.BlockSpec((NTOT, 1), lambda g: (0, 0)),
            pl.BlockSpec((1, NTOT), lambda g: (0, 0)),
            pl.BlockSpec((NTOT, 1), lambda g: (0, 0)),
        ],
        out_specs=[
            pl.BlockSpec((S, C), lambda g: (0, 0)),
            pl.BlockSpec((1, 1), lambda g: (0, 0)),
        ],
        out_shape=[
            jax.ShapeDtypeStruct((S, C), f32),
            jax.ShapeDtypeStruct((1, 1), f32),
        ],
        scratch_shapes=[
            pltpu.VMEM((NTOT, 128), f32),
            pltpu.VMEM((NTOT, 128), f32),
            pltpu.SMEM((1, 1), f32),
            pltpu.SMEM((1, 1), f32),
        ],
        compiler_params=pltpu.CompilerParams(
            dimension_semantics=("arbitrary",)),
    )(c0, c1, o0, o1, s_col, s_row, m_col)

    select0 = jnp.stack([d0, d1], axis=-1)
    return (out, select0, bal.reshape(()), dist.reshape(()))


# speedup vs baseline: 4.6731x; 1.1057x over previous
import functools

import jax
import jax.numpy as jnp
import numpy as np
from jax.experimental import pallas as pl
from jax.experimental.pallas import tpu as pltpu

S = 1024
D = 1024
E = 8
K = 2
C = 64
NTOT = 2 * S
BS = 920
N_REAL = 2 * BS
TI = 256
GSTEPS = NTOT // TI
H3 = 1000
HP = jax.lax.Precision.HIGHEST


def _const_parts(use_numpy):
    noise = jax.random.uniform(jax.random.key(1), (S, E), dtype=jnp.float32)
    k1 = jax.random.fold_in(jax.random.key(2), 0)
    k2 = jax.random.fold_in(jax.random.key(2), 1)
    s1 = jax.random.permutation(k1, S)[:BS]
    s2 = jax.random.permutation(k2, S)[:BS]
    if use_numpy:
        s1, s2 = np.asarray(s1), np.asarray(s2)
        w_src = np.zeros((S,), np.float32)
        w_src[s1] = 1.0
        w_tgt = np.zeros((S,), np.float32)
        w_tgt[s2] = 1.0
        m_flat = np.concatenate([w_src, w_tgt])
        s_flat = np.concatenate([w_src, -w_tgt])
        return np.asarray(noise), m_flat, s_flat
    w_src = jnp.zeros((S,), jnp.float32).at[s1].set(1.0)
    w_tgt = jnp.zeros((S,), jnp.float32).at[s2].set(1.0)
    m_flat = jnp.concatenate([w_src, w_tgt])
    s_flat = jnp.concatenate([w_src, -w_tgt])
    return noise, m_flat, s_flat


def _host_constants():
    try:
        cpu = jax.devices("cpu")[0]
        with jax.default_device(cpu):
            return _const_parts(use_numpy=True)
    except Exception:
        return None


_CONSTS = _host_constants()


def _route_body(x_ref, rwt_ref, noise_ref, c0_ref, c1_ref, d0_ref, d1_ref,
                bal_ref, ei_ref):
    x = x_ref[...]
    select = jax.lax.dot_general(x, rwt_ref[...], (((1,), (0,)), ((), ()))
                                 ) + noise_ref[...]
    lane = jax.lax.broadcasted_iota(jnp.int32, (S, E), 1)
    g0 = jnp.max(select, axis=1, keepdims=True)
    i0 = jnp.min(jnp.where(select == g0, lane, E), axis=1, keepdims=True)
    masked = jnp.where(lane == i0, -jnp.inf, select)
    g1 = jnp.max(masked, axis=1, keepdims=True)
    i1 = jnp.min(jnp.where(masked == g1, lane, E), axis=1, keepdims=True)
    m0 = (lane == i0).astype(jnp.float32)
    m1 = (lane == i1).astype(jnp.float32)
    c0 = g0 * m0
    c1 = g1 * m1
    d0 = (c0 != 0.0).astype(jnp.float32)
    d1 = (c1 != 0.0).astype(jnp.float32)
    c0_ref[...] = c0
    c1_ref[...] = c1
    d0_ref[...] = d0
    d1_ref[...] = d1
    density_colsum = jnp.sum((m0 + m1) * 0.5, axis=0, keepdims=True)
    proxy = jnp.sum(select, axis=0, keepdims=True) * (1.0 / S)
    bal = jnp.sum(proxy * density_colsum) * (float(E * E) / (S * E))
    bal_ref[...] = jnp.broadcast_to(bal, (1, 1))
    d01 = jnp.concatenate([d0, d1], axis=1)
    ei_ref[...] = jax.lax.dot_general(d01, x, (((0,), (0,)), ((), ())))


def _experts_body(ei_ref, w1_ref, b1_ref, w2_ref, b2_ref, w3_ref, b3_ref,
                  w4_ref, b4_ref, o_ref, h2_scr, h3_scr):
    j = pl.program_id(1)

    @pl.when(j == 0)
    def _front():
        inp = ei_ref[0]
        h = jax.lax.dot_general(inp, w1_ref[0], (((1,), (1,)), ((), ()))
                                ) + b1_ref[0]
        h = jnp.maximum(h, 0.0)
        h = jax.lax.dot_general(h, w2_ref[0], (((1,), (1,)), ((), ()))
                                ) + b2_ref[0]
        h2_scr[...] = jnp.maximum(h, 0.0)

    h3 = jax.lax.dot_general(h2_scr[...], w3_ref[0, 0],
                             (((1,), (1,)), ((), ()))) + b3_ref[0, 0]
    h3 = jnp.maximum(h3, 0.0)

    @pl.when(j == 0)
    def _store0():
        h3_scr[:, 0:H3] = h3

    @pl.when(j == 1)
    def _store1():
        h3_scr[:, H3:2 * H3] = h3
        o_ref[0] = jax.lax.dot_general(h3_scr[...], w4_ref[0],
                                       (((1,), (1,)), ((), ()))) + b4_ref[0]


def _mmd_body(c0_ref, c1_ref, o0_ref, o1_ref, scol_ref, srow_ref, mcol_ref,
              out_ref, dist_ref, a_scr, bt_scr, coef_scr, acc_scr):
    g = pl.program_id(0)

    @pl.when(g == 0)
    def _prep():
        mid0 = jax.lax.dot_general(c0_ref[...], o0_ref[...],
                                   (((1,), (0,)), ((), ())))
        mid1 = jax.lax.dot_general(c1_ref[...], o1_ref[...],
                                   (((1,), (0,)), ((), ())))
        out_ref[...] = mid0 + mid1
        n0 = jnp.sum(mid0 * mid0, axis=1, keepdims=True)
        n1 = jnp.sum(mid1 * mid1, axis=1, keepdims=True)
        ones = jnp.ones((S, 1), jnp.float32)
        zpad = jnp.zeros((S, 128 - C - 2), jnp.float32)
        a_scr[0:S, :] = jnp.concatenate([-2.0 * mid0, n0, ones, zpad], axis=1)
        a_scr[S:NTOT, :] = jnp.concatenate([-2.0 * mid1, n1, ones, zpad],
                                           axis=1)
        bt_scr[0:S, :] = jnp.concatenate([mid0, ones, n0, zpad], axis=1)
        bt_scr[S:NTOT, :] = jnp.concatenate([mid1, ones, n1, zpad], axis=1)
        m0c = mcol_ref[0:S, :]
        m1c = mcol_ref[S:NTOT, :]
        v = (jnp.sum(mid0 * m0c, axis=0, keepdims=True)
             + jnp.sum(mid1 * m1c, axis=0, keepdims=True))
        ssq = jnp.sum(v * v)
        sum_mn = jnp.sum(m0c * n0) + jnp.sum(m1c * n1)
        sum_l2 = 2.0 * N_REAL * sum_mn - 2.0 * ssq
        bw = sum_l2 / float(N_REAL * N_REAL - N_REAL)
        b0 = bw * 0.25
        coef_scr[0, 0] = 1.0 / (16.0 * b0)
        acc_scr[0, 0] = 0.0

    ablk = a_scr[pl.ds(g * TI, TI), :]
    l2 = jax.lax.dot_general(ablk, bt_scr[...], (((1,), (1,)), ((), ())))
    u = jnp.exp(-l2 * coef_scr[0, 0])
    u2 = u * u
    u4 = u2 * u2
    u8 = u4 * u4
    p = u + u2 + u4 + u8 + u8 * u8
    rs = jnp.sum(p * srow_ref[...], axis=1, keepdims=True)
    acc_scr[0, 0] += jnp.sum(rs * scol_ref[pl.ds(g * TI, TI), :])

    @pl.when(g == GSTEPS - 1)
    def _fin():
        dist_ref[...] = jnp.broadcast_to(-acc_scr[0, 0] / float(BS * BS),
                                         (1, 1))


@functools.partial(jax.jit, static_argnums=())
def kernel(x, router_w, W1, b1, W2, b2, W3, b3, W4, b4):
    f32 = jnp.float32
    noise, m_flat, s_flat = (_CONSTS if _CONSTS is not None
                             else _const_parts(use_numpy=False))
    noise = jnp.asarray(noise)
    m_col = jnp.asarray(m_flat).reshape(NTOT, 1)
    s_col = jnp.asarray(s_flat).reshape(NTOT, 1)
    s_row = jnp.asarray(s_flat).reshape(1, NTOT)

    c0, c1, d0, d1, bal, ei = pl.pallas_call(
        _route_body,
        out_shape=[
            jax.ShapeDtypeStruct((S, E), f32),
            jax.ShapeDtypeStruct((S, E), f32),
            jax.ShapeDtypeStruct((S, E), f32),
            jax.ShapeDtypeStruct((S, E), f32),
            jax.ShapeDtypeStruct((1, 1), f32),
            jax.ShapeDtypeStruct((2 * E, D), f32),
        ],
    )(x, router_w.T, noise)

    eik = jnp.stack([ei[:E], ei[E:]], axis=1)
    outs = pl.pallas_call(
        _experts_body,
        grid=(E, 2),
        in_specs=[
            pl.BlockSpec((1, K, D), lambda e, j: (e, 0, 0)),
            pl.BlockSpec((1, 500, D), lambda e, j: (e, 0, 0)),
            pl.BlockSpec((1, 1, 500), lambda e, j: (e, 0, 0)),
            pl.BlockSpec((1, 500, 500), lambda e, j: (e, 0, 0)),
            pl.BlockSpec((1, 1, 500), lambda e, j: (e, 0, 0)),
            pl.BlockSpec((1, 1, H3, 500), lambda e, j: (e, j, 0, 0)),
            pl.BlockSpec((1, 1, 1, H3), lambda e, j: (e, j, 0, 0)),
            pl.BlockSpec((1, C, 2 * H3), lambda e, j: (e, 0, 0)),
            pl.BlockSpec((1, 1, C), lambda e, j: (e, 0, 0)),
        ],
        out_specs=pl.BlockSpec((1, K, C), lambda e, j: (e, 0, 0)),
        out_shape=jax.ShapeDtypeStruct((E, K, C), f32),
        scratch_shapes=[pltpu.VMEM((K, 500), f32),
                        pltpu.VMEM((K, 2 * H3), f32)],
        compiler_params=pltpu.CompilerParams(
            dimension_semantics=("arbitrary", "arbitrary")),
    )(eik, W1, b1.reshape(E, 1, 500), W2, b2.reshape(E, 1, 500),
      W3.reshape(E, 2, H3, 500), b3.reshape(E, 2, 1, H3),
      W4, b4.reshape(E, 1, C))

    o0 = outs[:, 0, :]
    o1 = outs[:, 1, :]
    out, dist = pl.pallas_call(
        _mmd_body,
        grid=(GSTEPS,),
        in_specs=[
            pl.BlockSpec((S, E), lambda g: (0, 0)),
            pl.BlockSpec((S, E), lambda g: (0, 0)),
            pl.BlockSpec((E, C), lambda g: (0, 0)),
            pl.BlockSpec((E, C), lambda g: (0, 0)),
            pl.BlockSpec((NTOT, 1), lambda g: (0, 0)),
            pl.BlockSpec((1, NTOT), lambda g: (0, 0)),
            pl.BlockSpec((NTOT, 1), lambda g: (0, 0)),
        ],
        out_specs=[
            pl.BlockSpec((S, C), lambda g: (0, 0)),
            pl.BlockSpec((1, 1), lambda g: (0, 0)),
        ],
        out_shape=[
            jax.ShapeDtypeStruct((S, C), f32),
            jax.ShapeDtypeStruct((1, 1), f32),
        ],
        scratch_shapes=[
            pltpu.VMEM((NTOT, 128), f32),
            pltpu.VMEM((NTOT, 128), f32),
            pltpu.SMEM((1, 1), f32),
            pltpu.SMEM((1, 1), f32),
        ],
        compiler_params=pltpu.CompilerParams(
            dimension_semantics=("arbitrary",)),
    )(c0, c1, o0, o1, s_col, s_row, m_col)

    select0 = jnp.stack([d0, d1], axis=-1)
    return (out, select0, bal.reshape(()), dist.reshape(()))
